# baseline (speedup 1.0000x reference)
import functools

import jax
import jax.numpy as jnp
from jax import lax
from jax.experimental import pallas as pl
from jax.experimental.pallas import tpu as pltpu
from jax.experimental.pallas import tpu_sc as plsc

N = 10000
E = 320000
IN_CH = 128
HID = 16
OUT_CH = 128

NC = 2
NS = 16
NW = NC * NS
EB = 128
KB = 80
E_PAD = NW * KB * EB
N_PAD = 10240
RPS = N_PAD // NS
ZCH = RPS // EB
NBUF = 8
NCH = KB // NBUF

_mesh = plsc.VectorSubcoreMesh(core_axis_name="c", subcore_axis_name="s")


def _fill_rows(buf, nrows, vec):
    def body(i, _):
        buf[i, :] = vec
        return 0

    lax.fori_loop(0, nrows, body, 0)


@functools.partial(
    pl.kernel,
    out_type=jax.ShapeDtypeStruct((NC, N_PAD, HID), jnp.float32),
    mesh=_mesh,
    scratch_types=[
        pltpu.VMEM((KB, EB), jnp.int32),
        pltpu.VMEM((EB, HID), jnp.float32),
        pltpu.VMEM_SHARED((N_PAD, HID), jnp.float32),
        pltpu.SemaphoreType.DMA,
    ],
    compiler_params=pltpu.CompilerParams(use_tc_tiling_on_sc=False),
)
def _deg_kernel(dst_hbm, out_hbm, dst_v, ones_v, acc, sem):
    cid = lax.axis_index("c")
    sid = lax.axis_index("s")
    wid = cid * NS + sid

    _fill_rows(ones_v, EB, jnp.zeros((16,), jnp.float32))
    for t in range(ZCH):
        pltpu.sync_copy(ones_v, acc.at[pl.ds(sid * RPS + t * EB, EB)])
    _fill_rows(ones_v, EB, jnp.ones((16,), jnp.float32))
    pltpu.sync_copy(dst_hbm.at[wid], dst_v)
    plsc.subcore_barrier()

    def edge_block(j, _):
        @pl.when(j >= 16)
        def _():
            pltpu.make_async_copy(ones_v, acc.at[dst_v.at[j - 16]], sem).wait()

        pltpu.async_copy(ones_v, acc.at[dst_v.at[j]], sem, add=True)
        return 0

    lax.fori_loop(0, KB, edge_block, 0)
    for t in range(16):
        pltpu.make_async_copy(ones_v, acc.at[dst_v.at[KB - 16 + t]], sem).wait()
    plsc.subcore_barrier()
    pltpu.sync_copy(acc.at[pl.ds(sid * RPS, RPS)],
                    out_hbm.at[cid, pl.ds(sid * RPS, RPS)])


def _newton_rsqrt(d):
    ibits = lax.bitcast_convert_type(d, jnp.int32)
    seed = jnp.int32(0x5F3759DF) - lax.shift_right_logical(ibits, 1)
    y = lax.bitcast_convert_type(seed, jnp.float32)
    for _ in range(3):
        y = y * (1.5 - 0.5 * d * y * y)
    return y


@functools.partial(
    pl.kernel,
    out_type=[
        jax.ShapeDtypeStruct((NC, N_PAD, HID), jnp.float32),
        jax.ShapeDtypeStruct((N_PAD, HID), jnp.float32),
    ],
    mesh=_mesh,
    scratch_types=[
        pltpu.VMEM((KB, EB), jnp.int32),
        pltpu.VMEM((KB, EB), jnp.int32),
        pltpu.VMEM((2, NBUF, EB, HID), jnp.float32),
        pltpu.VMEM((EB, HID), jnp.float32),
        pltpu.VMEM((RPS, HID), jnp.float32),
        pltpu.VMEM((RPS, HID), jnp.float32),
        pltpu.VMEM((RPS, HID), jnp.float32),
        pltpu.VMEM_SHARED((N_PAD, HID), jnp.float32),
        pltpu.VMEM_SHARED((N_PAD, HID), jnp.float32),
        pltpu.SemaphoreType.DMA,
        pltpu.SemaphoreType.DMA,
        pltpu.SemaphoreType.DMA,
        pltpu.SemaphoreType.DMA,
    ],
    compiler_params=pltpu.CompilerParams(use_tc_tiling_on_sc=False),
)
def _agg1_kernel(src_hbm, dst_hbm, deg_hbm, h1_hbm, out_hbm, g1_hbm,
                 src_v, dst_v, rows_v, zero_v, d0_v, d1_v, g_v,
                 acc, g_sp, gs0, gs1, ss0, ss1):
    cid = lax.axis_index("c")
    sid = lax.axis_index("s")
    wid = cid * NS + sid
    row0 = sid * RPS

    pltpu.sync_copy(deg_hbm.at[0, pl.ds(row0, RPS)], d0_v)
    pltpu.sync_copy(deg_hbm.at[1, pl.ds(row0, RPS)], d1_v)
    pltpu.sync_copy(h1_hbm.at[pl.ds(row0, RPS)], g_v)

    def row_body(i, _):
        d = d0_v[i, :] + d1_v[i, :] + 1.0
        g_v[i, :] = _newton_rsqrt(d) * g_v[i, :]
        return 0

    lax.fori_loop(0, RPS, row_body, 0)
    pltpu.sync_copy(g_v, g_sp.at[pl.ds(row0, RPS)])

    @pl.when(cid == 0)
    def _():
        pltpu.sync_copy(g_v, g1_hbm.at[pl.ds(row0, RPS)])

    _fill_rows(zero_v, EB, jnp.zeros((16,), jnp.float32))
    for t in range(ZCH):
        pltpu.sync_copy(zero_v, acc.at[pl.ds(row0 + t * EB, EB)])
    pltpu.sync_copy(src_hbm.at[wid], src_v)
    pltpu.sync_copy(dst_hbm.at[wid], dst_v)
    plsc.subcore_barrier()
    _edge_pipeline(src_v, dst_v, rows_v, acc, g_sp, gs0, gs1, ss0, ss1)
    plsc.subcore_barrier()
    pltpu.sync_copy(acc.at[pl.ds(row0, RPS)],
                    out_hbm.at[cid, pl.ds(row0, RPS)])


def _edge_pipeline(src_v, dst_v, rows_v, acc, g_sp, gs0, gs1, ss0, ss1):
    def gather(c, p, sem):
        for b in range(NBUF):
            pltpu.async_copy(g_sp.at[src_v.at[c * NBUF + b]],
                             rows_v.at[p, b], sem)

    def wait_gather(c, p, sem):
        for b in range(NBUF):
            pltpu.make_async_copy(g_sp.at[src_v.at[c * NBUF + b]],
                                  rows_v.at[p, b], sem).wait()

    def scatter(c, p, sem):
        for b in range(NBUF):
            pltpu.async_copy(rows_v.at[p, b],
                             acc.at[dst_v.at[c * NBUF + b]], sem, add=True)

    def wait_scatter(c, p, sem):
        for b in range(NBUF):
            pltpu.make_async_copy(rows_v.at[p, b],
                                  acc.at[dst_v.at[c * NBUF + b]], sem).wait()

    gather(0, 0, gs0)

    def body(i, _):
        c0 = 2 * i
        c1 = c0 + 1

        @pl.when(i > 0)
        def _():
            wait_scatter(c0 - 1, 1, ss1)

        gather(c1, 1, gs1)
        wait_gather(c0, 0, gs0)
        scatter(c0, 0, ss0)
        wait_gather(c1, 1, gs1)
        scatter(c1, 1, ss1)
        wait_scatter(c0, 0, ss0)

        @pl.when(i < NCH // 2 - 1)
        def _():
            gather(c0 + 2, 0, gs0)

        return 0

    lax.fori_loop(0, NCH // 2, body, 0)
    wait_scatter(NCH - 1, 1, ss1)


@functools.partial(
    pl.kernel,
    out_type=[
        jax.ShapeDtypeStruct((NC, N_PAD, HID), jnp.float32),
        jax.ShapeDtypeStruct((N_PAD, HID), jnp.float32),
    ],
    mesh=_mesh,
    scratch_types=[
        pltpu.VMEM((KB, EB), jnp.int32),
        pltpu.VMEM((KB, EB), jnp.int32),
        pltpu.VMEM((2, NBUF, EB, HID), jnp.float32),
        pltpu.VMEM((EB, HID), jnp.float32),
        pltpu.VMEM((RPS, HID), jnp.float32),
        pltpu.VMEM((RPS, HID), jnp.float32),
        pltpu.VMEM((RPS, HID), jnp.float32),
        pltpu.VMEM((RPS, HID), jnp.float32),
        pltpu.VMEM((RPS, HID), jnp.float32),
        pltpu.VMEM((1, HID), jnp.float32),
        pltpu.VMEM_SHARED((N_PAD, HID), jnp.float32),
        pltpu.VMEM_SHARED((N_PAD, HID), jnp.float32),
        pltpu.SemaphoreType.DMA,
        pltpu.SemaphoreType.DMA,
        pltpu.SemaphoreType.DMA,
        pltpu.SemaphoreType.DMA,
    ],
    compiler_params=pltpu.CompilerParams(use_tc_tiling_on_sc=False),
)
def _agg2_kernel(src_hbm, dst_hbm, deg_hbm, s1_hbm, g1_hbm, b1_hbm,
                 out_hbm, g2_hbm,
                 src_v, dst_v, rows_v, zero_v, d0_v, d1_v, s0_v, s1_v, g_v,
                 b1_v, acc, g_sp, gs0, gs1, ss0, ss1):
    cid = lax.axis_index("c")
    sid = lax.axis_index("s")
    wid = cid * NS + sid
    row0 = sid * RPS

    pltpu.sync_copy(deg_hbm.at[0, pl.ds(row0, RPS)], d0_v)
    pltpu.sync_copy(deg_hbm.at[1, pl.ds(row0, RPS)], d1_v)
    pltpu.sync_copy(s1_hbm.at[0, pl.ds(row0, RPS)], s0_v)
    pltpu.sync_copy(s1_hbm.at[1, pl.ds(row0, RPS)], s1_v)
    pltpu.sync_copy(g1_hbm.at[pl.ds(row0, RPS)], g_v)
    pltpu.sync_copy(b1_hbm, b1_v)
    bias = b1_v[0, :]

    def row_body(i, _):
        d = d0_v[i, :] + d1_v[i, :] + 1.0
        dinv = _newton_rsqrt(d)
        s = s0_v[i, :] + s1_v[i, :] + g_v[i, :]
        h = jnp.maximum(dinv * s + bias, 0.0)
        g_v[i, :] = dinv * h
        return 0

    lax.fori_loop(0, RPS, row_body, 0)
    pltpu.sync_copy(g_v, g_sp.at[pl.ds(row0, RPS)])

    @pl.when(cid == 0)
    def _():
        pltpu.sync_copy(g_v, g2_hbm.at[pl.ds(row0, RPS)])

    _fill_rows(zero_v, EB, jnp.zeros((16,), jnp.float32))
    for t in range(ZCH):
        pltpu.sync_copy(zero_v, acc.at[pl.ds(row0 + t * EB, EB)])
    pltpu.sync_copy(src_hbm.at[wid], src_v)
    pltpu.sync_copy(dst_hbm.at[wid], dst_v)
    plsc.subcore_barrier()
    _edge_pipeline(src_v, dst_v, rows_v, acc, g_sp, gs0, gs1, ss0, ss1)
    plsc.subcore_barrier()
    pltpu.sync_copy(acc.at[pl.ds(row0, RPS)],
                    out_hbm.at[cid, pl.ds(row0, RPS)])


_R = 2048
_R3 = 2000


def _tc_mm_body(x_ref, w1_ref, h1_ref):
    h1_ref[...] = jnp.dot(x_ref[...], w1_ref[...],
                          preferred_element_type=jnp.float32)


def _tc_mm(x, W1):
    return pl.pallas_call(
        _tc_mm_body,
        grid=(N_PAD // _R,),
        in_specs=[
            pl.BlockSpec((_R, IN_CH), lambda i: (i, 0)),
            pl.BlockSpec((IN_CH, HID), lambda i: (0, 0)),
        ],
        out_specs=pl.BlockSpec((_R, HID), lambda i: (i, 0)),
        out_shape=jax.ShapeDtypeStruct((N_PAD, HID), jnp.float32),
    )(x, W1)


def _tc3_body(s0_ref, s1_ref, g2_ref, d0_ref, d1_ref, w2_ref, b2_ref, out_ref):
    dinv = lax.rsqrt(d0_ref[...] + d1_ref[...] + 1.0)
    agg = dinv * (s0_ref[...] + s1_ref[...] + g2_ref[...])
    o = jnp.dot(agg, w2_ref[...], preferred_element_type=jnp.float32)
    o = o + b2_ref[...]
    m = jnp.max(o, axis=1, keepdims=True)
    lse = m + jnp.log(jnp.sum(jnp.exp(o - m), axis=1, keepdims=True))
    out_ref[...] = o - lse


def _tc3(s0, s1, g2, d0, d1, W2, b2):
    return pl.pallas_call(
        _tc3_body,
        grid=(N // _R3,),
        in_specs=[
            pl.BlockSpec((_R3, HID), lambda i: (i, 0)),
            pl.BlockSpec((_R3, HID), lambda i: (i, 0)),
            pl.BlockSpec((_R3, HID), lambda i: (i, 0)),
            pl.BlockSpec((_R3, HID), lambda i: (i, 0)),
            pl.BlockSpec((_R3, HID), lambda i: (i, 0)),
            pl.BlockSpec((HID, OUT_CH), lambda i: (0, 0)),
            pl.BlockSpec((1, OUT_CH), lambda i: (0, 0)),
        ],
        out_specs=pl.BlockSpec((_R3, OUT_CH), lambda i: (i, 0)),
        out_shape=jax.ShapeDtypeStruct((N, OUT_CH), jnp.float32),
    )(s0, s1, g2, d0, d1, W2, b2)


def kernel(x, edge_index, W1, b1, W2, b2):
    pad = jnp.full((E_PAD - E,), N, jnp.int32)
    src = jnp.concatenate([edge_index[0].astype(jnp.int32), pad])
    dst = jnp.concatenate([edge_index[1].astype(jnp.int32), pad])
    src = src.reshape(NW, KB, EB)
    dst = dst.reshape(NW, KB, EB)
    xp = jnp.zeros((N_PAD, IN_CH), x.dtype).at[:N].set(x)
    b1r = b1.reshape(1, HID)
    b2r = b2.reshape(1, OUT_CH)

    degp = _deg_kernel(dst)
    h1 = _tc_mm(xp, W1)
    s1p, g1 = _agg1_kernel(src, dst, degp, h1)
    s2p, g2 = _agg2_kernel(src, dst, degp, s1p, g1, b1r)
    return _tc3(s2p[0], s2p[1], g2, degp[0], degp[1], W2, b2r)


# speedup vs baseline: 1.1050x; 1.1050x over previous
import functools

import jax
import jax.numpy as jnp
from jax import lax
from jax.experimental import pallas as pl
from jax.experimental.pallas import tpu as pltpu
from jax.experimental.pallas import tpu_sc as plsc

N = 10000
E = 320000
IN_CH = 128
HID = 16
OUT_CH = 128

NC = 2
NS = 16
NW = NC * NS
EB = 128
KB = 80
E_PAD = NW * KB * EB
N_PAD = 10240
RPS = N_PAD // NS
ZCH = RPS // EB
NBUF = 4
NCH = KB // NBUF

_mesh = plsc.VectorSubcoreMesh(core_axis_name="c", subcore_axis_name="s")


def _fill_rows(buf, nrows, vec):
    def body(i, _):
        buf[i, :] = vec
        return 0

    lax.fori_loop(0, nrows, body, 0)


@functools.partial(
    pl.kernel,
    out_type=jax.ShapeDtypeStruct((NC, N_PAD, HID), jnp.float32),
    mesh=_mesh,
    scratch_types=[
        pltpu.VMEM((KB, EB), jnp.int32),
        pltpu.VMEM((EB, HID), jnp.float32),
        pltpu.VMEM((EB, HID), jnp.float32),
        pltpu.VMEM_SHARED((N_PAD, HID), jnp.float32),
        pltpu.SemaphoreType.DMA,
        pltpu.SemaphoreType.DMA,
    ],
    compiler_params=pltpu.CompilerParams(use_tc_tiling_on_sc=False),
)
def _deg_kernel(dst_hbm, out_hbm, dst_v, ones_v, zero_v, acc, sem, sem2):
    cid = lax.axis_index("c")
    sid = lax.axis_index("s")
    wid = cid * NS + sid

    pltpu.async_copy(dst_hbm.at[wid], dst_v, sem2)
    _fill_rows(zero_v, EB, jnp.zeros((16,), jnp.float32))
    for t in range(ZCH):
        pltpu.async_copy(zero_v, acc.at[pl.ds(sid * RPS + t * EB, EB)], sem)
    _fill_rows(ones_v, EB, jnp.ones((16,), jnp.float32))
    for t in range(ZCH):
        pltpu.make_async_copy(zero_v, acc.at[pl.ds(sid * RPS + t * EB, EB)],
                              sem).wait()
    pltpu.make_async_copy(dst_hbm.at[wid], dst_v, sem2).wait()
    plsc.subcore_barrier()

    def edge_block(j, _):
        @pl.when(j >= 16)
        def _():
            pltpu.make_async_copy(ones_v, acc.at[dst_v.at[j - 16]], sem).wait()

        pltpu.async_copy(ones_v, acc.at[dst_v.at[j]], sem, add=True)
        return 0

    lax.fori_loop(0, KB, edge_block, 0)
    for t in range(16):
        pltpu.make_async_copy(ones_v, acc.at[dst_v.at[KB - 16 + t]], sem).wait()
    plsc.subcore_barrier()
    pltpu.sync_copy(acc.at[pl.ds(sid * RPS, RPS)],
                    out_hbm.at[cid, pl.ds(sid * RPS, RPS)])


def _newton_rsqrt(d):
    ibits = lax.bitcast_convert_type(d, jnp.int32)
    seed = jnp.int32(0x5F3759DF) - lax.shift_right_logical(ibits, 1)
    y = lax.bitcast_convert_type(seed, jnp.float32)
    for _ in range(3):
        y = y * (1.5 - 0.5 * d * y * y)
    return y


@functools.partial(
    pl.kernel,
    out_type=[
        jax.ShapeDtypeStruct((NC, N_PAD, HID), jnp.float32),
        jax.ShapeDtypeStruct((N_PAD, HID), jnp.float32),
    ],
    mesh=_mesh,
    scratch_types=[
        pltpu.VMEM((KB, EB), jnp.int32),
        pltpu.VMEM((KB, EB), jnp.int32),
        pltpu.VMEM((2, NBUF, EB, HID), jnp.float32),
        pltpu.VMEM((EB, HID), jnp.float32),
        pltpu.VMEM((RPS, HID), jnp.float32),
        pltpu.VMEM((RPS, HID), jnp.float32),
        pltpu.VMEM((RPS, HID), jnp.float32),
        pltpu.VMEM_SHARED((N_PAD, HID), jnp.float32),
        pltpu.VMEM_SHARED((N_PAD, HID), jnp.float32),
        pltpu.SemaphoreType.DMA,
        pltpu.SemaphoreType.DMA,
        pltpu.SemaphoreType.DMA,
        pltpu.SemaphoreType.DMA,
    ],
    compiler_params=pltpu.CompilerParams(use_tc_tiling_on_sc=False),
)
def _agg1_kernel(src_hbm, dst_hbm, deg_hbm, h1_hbm, out_hbm, g1_hbm,
                 src_v, dst_v, rows_v, zero_v, d0_v, d1_v, g_v,
                 acc, g_sp, gs0, gs1, ss0, ss1):
    cid = lax.axis_index("c")
    sid = lax.axis_index("s")
    wid = cid * NS + sid
    row0 = sid * RPS

    pltpu.async_copy(deg_hbm.at[0, pl.ds(row0, RPS)], d0_v, gs0)
    pltpu.async_copy(deg_hbm.at[1, pl.ds(row0, RPS)], d1_v, gs0)
    pltpu.async_copy(h1_hbm.at[pl.ds(row0, RPS)], g_v, gs0)
    pltpu.async_copy(src_hbm.at[wid], src_v, gs1)
    pltpu.async_copy(dst_hbm.at[wid], dst_v, gs1)
    _fill_rows(zero_v, EB, jnp.zeros((16,), jnp.float32))
    for t in range(ZCH):
        pltpu.async_copy(zero_v, acc.at[pl.ds(row0 + t * EB, EB)], ss0)
    pltpu.make_async_copy(deg_hbm.at[0, pl.ds(row0, RPS)], d0_v, gs0).wait()
    pltpu.make_async_copy(deg_hbm.at[1, pl.ds(row0, RPS)], d1_v, gs0).wait()
    pltpu.make_async_copy(h1_hbm.at[pl.ds(row0, RPS)], g_v, gs0).wait()

    def row_body(i, _):
        d = d0_v[i, :] + d1_v[i, :] + 1.0
        g_v[i, :] = _newton_rsqrt(d) * g_v[i, :]
        return 0

    lax.fori_loop(0, RPS, row_body, 0)
    pltpu.sync_copy(g_v, g_sp.at[pl.ds(row0, RPS)])

    @pl.when(cid == 0)
    def _():
        pltpu.sync_copy(g_v, g1_hbm.at[pl.ds(row0, RPS)])

    pltpu.make_async_copy(src_hbm.at[wid], src_v, gs1).wait()
    pltpu.make_async_copy(dst_hbm.at[wid], dst_v, gs1).wait()
    for t in range(ZCH):
        pltpu.make_async_copy(zero_v, acc.at[pl.ds(row0 + t * EB, EB)],
                              ss0).wait()
    plsc.subcore_barrier()
    _edge_pipeline(src_v, dst_v, rows_v, acc, g_sp, gs0, gs1, ss0, ss1)
    plsc.subcore_barrier()
    pltpu.sync_copy(acc.at[pl.ds(row0, RPS)],
                    out_hbm.at[cid, pl.ds(row0, RPS)])


def _edge_pipeline(src_v, dst_v, rows_v, acc, g_sp, gs0, gs1, ss0, ss1):
    def gather(c, p, sem):
        for b in range(NBUF):
            pltpu.async_copy(g_sp.at[src_v.at[c * NBUF + b]],
                             rows_v.at[p, b], sem)

    def wait_gather(c, p, sem):
        for b in range(NBUF):
            pltpu.make_async_copy(g_sp.at[src_v.at[c * NBUF + b]],
                                  rows_v.at[p, b], sem).wait()

    def scatter(c, p, sem):
        for b in range(NBUF):
            pltpu.async_copy(rows_v.at[p, b],
                             acc.at[dst_v.at[c * NBUF + b]], sem, add=True)

    def wait_scatter(c, p, sem):
        for b in range(NBUF):
            pltpu.make_async_copy(rows_v.at[p, b],
                                  acc.at[dst_v.at[c * NBUF + b]], sem).wait()

    gather(0, 0, gs0)

    def body(i, _):
        c0 = 2 * i
        c1 = c0 + 1

        @pl.when(i > 0)
        def _():
            wait_scatter(c0 - 1, 1, ss1)

        gather(c1, 1, gs1)
        wait_gather(c0, 0, gs0)
        scatter(c0, 0, ss0)
        wait_gather(c1, 1, gs1)
        scatter(c1, 1, ss1)
        wait_scatter(c0, 0, ss0)

        @pl.when(i < NCH // 2 - 1)
        def _():
            gather(c0 + 2, 0, gs0)

        return 0

    lax.fori_loop(0, NCH // 2, body, 0)
    wait_scatter(NCH - 1, 1, ss1)


@functools.partial(
    pl.kernel,
    out_type=[
        jax.ShapeDtypeStruct((NC, N_PAD, HID), jnp.float32),
        jax.ShapeDtypeStruct((N_PAD, HID), jnp.float32),
    ],
    mesh=_mesh,
    scratch_types=[
        pltpu.VMEM((KB, EB), jnp.int32),
        pltpu.VMEM((KB, EB), jnp.int32),
        pltpu.VMEM((2, NBUF, EB, HID), jnp.float32),
        pltpu.VMEM((EB, HID), jnp.float32),
        pltpu.VMEM((RPS, HID), jnp.float32),
        pltpu.VMEM((RPS, HID), jnp.float32),
        pltpu.VMEM((RPS, HID), jnp.float32),
        pltpu.VMEM((RPS, HID), jnp.float32),
        pltpu.VMEM((RPS, HID), jnp.float32),
        pltpu.VMEM((1, HID), jnp.float32),
        pltpu.VMEM_SHARED((N_PAD, HID), jnp.float32),
        pltpu.VMEM_SHARED((N_PAD, HID), jnp.float32),
        pltpu.SemaphoreType.DMA,
        pltpu.SemaphoreType.DMA,
        pltpu.SemaphoreType.DMA,
        pltpu.SemaphoreType.DMA,
    ],
    compiler_params=pltpu.CompilerParams(use_tc_tiling_on_sc=False),
)
def _agg2_kernel(src_hbm, dst_hbm, deg_hbm, s1_hbm, g1_hbm, b1_hbm,
                 out_hbm, g2_hbm,
                 src_v, dst_v, rows_v, zero_v, d0_v, d1_v, s0_v, s1_v, g_v,
                 b1_v, acc, g_sp, gs0, gs1, ss0, ss1):
    cid = lax.axis_index("c")
    sid = lax.axis_index("s")
    wid = cid * NS + sid
    row0 = sid * RPS

    pltpu.async_copy(deg_hbm.at[0, pl.ds(row0, RPS)], d0_v, gs0)
    pltpu.async_copy(deg_hbm.at[1, pl.ds(row0, RPS)], d1_v, gs0)
    pltpu.async_copy(s1_hbm.at[0, pl.ds(row0, RPS)], s0_v, gs0)
    pltpu.async_copy(s1_hbm.at[1, pl.ds(row0, RPS)], s1_v, gs0)
    pltpu.async_copy(g1_hbm.at[pl.ds(row0, RPS)], g_v, gs0)
    pltpu.async_copy(b1_hbm, b1_v, gs0)
    pltpu.async_copy(src_hbm.at[wid], src_v, gs1)
    pltpu.async_copy(dst_hbm.at[wid], dst_v, gs1)
    _fill_rows(zero_v, EB, jnp.zeros((16,), jnp.float32))
    for t in range(ZCH):
        pltpu.async_copy(zero_v, acc.at[pl.ds(row0 + t * EB, EB)], ss0)
    pltpu.make_async_copy(deg_hbm.at[0, pl.ds(row0, RPS)], d0_v, gs0).wait()
    pltpu.make_async_copy(deg_hbm.at[1, pl.ds(row0, RPS)], d1_v, gs0).wait()
    pltpu.make_async_copy(s1_hbm.at[0, pl.ds(row0, RPS)], s0_v, gs0).wait()
    pltpu.make_async_copy(s1_hbm.at[1, pl.ds(row0, RPS)], s1_v, gs0).wait()
    pltpu.make_async_copy(g1_hbm.at[pl.ds(row0, RPS)], g_v, gs0).wait()
    pltpu.make_async_copy(b1_hbm, b1_v, gs0).wait()
    bias = b1_v[0, :]

    def row_body(i, _):
        d = d0_v[i, :] + d1_v[i, :] + 1.0
        dinv = _newton_rsqrt(d)
        s = s0_v[i, :] + s1_v[i, :] + g_v[i, :]
        h = jnp.maximum(dinv * s + bias, 0.0)
        g_v[i, :] = dinv * h
        return 0

    lax.fori_loop(0, RPS, row_body, 0)
    pltpu.sync_copy(g_v, g_sp.at[pl.ds(row0, RPS)])

    @pl.when(cid == 0)
    def _():
        pltpu.sync_copy(g_v, g2_hbm.at[pl.ds(row0, RPS)])

    pltpu.make_async_copy(src_hbm.at[wid], src_v, gs1).wait()
    pltpu.make_async_copy(dst_hbm.at[wid], dst_v, gs1).wait()
    for t in range(ZCH):
        pltpu.make_async_copy(zero_v, acc.at[pl.ds(row0 + t * EB, EB)],
                              ss0).wait()
    plsc.subcore_barrier()
    _edge_pipeline(src_v, dst_v, rows_v, acc, g_sp, gs0, gs1, ss0, ss1)
    plsc.subcore_barrier()
    pltpu.sync_copy(acc.at[pl.ds(row0, RPS)],
                    out_hbm.at[cid, pl.ds(row0, RPS)])


_R = 2048
_R3 = 2000


def _tc_mm_body(x_ref, w1_ref, h1_ref):
    h1_ref[...] = jnp.dot(x_ref[...], w1_ref[...],
                          preferred_element_type=jnp.float32)


def _tc_mm(x, W1):
    return pl.pallas_call(
        _tc_mm_body,
        grid=(N_PAD // _R,),
        in_specs=[
            pl.BlockSpec((_R, IN_CH), lambda i: (i, 0)),
            pl.BlockSpec((IN_CH, HID), lambda i: (0, 0)),
        ],
        out_specs=pl.BlockSpec((_R, HID), lambda i: (i, 0)),
        out_shape=jax.ShapeDtypeStruct((N_PAD, HID), jnp.float32),
    )(x, W1)


def _tc3_body(s0_ref, s1_ref, g2_ref, d0_ref, d1_ref, w2_ref, b2_ref, out_ref):
    dinv = lax.rsqrt(d0_ref[...] + d1_ref[...] + 1.0)
    agg = dinv * (s0_ref[...] + s1_ref[...] + g2_ref[...])
    o = jnp.dot(agg, w2_ref[...], preferred_element_type=jnp.float32)
    o = o + b2_ref[...]
    m = jnp.max(o, axis=1, keepdims=True)
    lse = m + jnp.log(jnp.sum(jnp.exp(o - m), axis=1, keepdims=True))
    out_ref[...] = o - lse


def _tc3(s0, s1, g2, d0, d1, W2, b2):
    return pl.pallas_call(
        _tc3_body,
        grid=(N // _R3,),
        in_specs=[
            pl.BlockSpec((_R3, HID), lambda i: (i, 0)),
            pl.BlockSpec((_R3, HID), lambda i: (i, 0)),
            pl.BlockSpec((_R3, HID), lambda i: (i, 0)),
            pl.BlockSpec((_R3, HID), lambda i: (i, 0)),
            pl.BlockSpec((_R3, HID), lambda i: (i, 0)),
            pl.BlockSpec((HID, OUT_CH), lambda i: (0, 0)),
            pl.BlockSpec((1, OUT_CH), lambda i: (0, 0)),
        ],
        out_specs=pl.BlockSpec((_R3, OUT_CH), lambda i: (i, 0)),
        out_shape=jax.ShapeDtypeStruct((N, OUT_CH), jnp.float32),
    )(s0, s1, g2, d0, d1, W2, b2)


def kernel(x, edge_index, W1, b1, W2, b2):
    pad = jnp.full((E_PAD - E,), N, jnp.int32)
    src = jnp.concatenate([edge_index[0].astype(jnp.int32), pad])
    dst = jnp.concatenate([edge_index[1].astype(jnp.int32), pad])
    src = src.reshape(NW, KB, EB)
    dst = dst.reshape(NW, KB, EB)
    xp = jnp.zeros((N_PAD, IN_CH), x.dtype).at[:N].set(x)
    b1r = b1.reshape(1, HID)
    b2r = b2.reshape(1, OUT_CH)

    degp = _deg_kernel(dst)
    h1 = _tc_mm(xp, W1)
    s1p, g1 = _agg1_kernel(src, dst, degp, h1)
    s2p, g2 = _agg2_kernel(src, dst, degp, s1p, g1, b1r)
    return _tc3(s2p[0], s2p[1], g2, degp[0], degp[1], W2, b2r)
